# Initial kernel scaffold; baseline (speedup 1.0000x reference)
import jax
import jax.numpy as jnp
from jax.experimental import pallas as pl


def kernel(desc0, desc1, W0, q0, k0, cb0, lw0, lb0, gam0, bet0, W1, q1, k1, cb1, lw1, lb1, gam1, bet1):
    raise NotImplementedError("write your pallas kernel here")


# speedup vs baseline: 1554.2668x; 1554.2668x over previous
import functools

import jax
import jax.numpy as jnp
from jax.experimental import pallas as pl

B = 2
S0 = 256
S1 = 256
N = S0 + S1
F = 128
BN = B * N
NEG_SLOPE = 0.2


def _dot(a, b):
    return jax.lax.dot_general(
        a, b, (((1,), (0,)), ((), ())),
        precision=jax.lax.Precision.HIGHEST,
        preferred_element_type=jnp.float32)


def _dense_layer(x, wa, wb, q, k, cb, lwa, lwb, lb, gam, bet, rel1f, inval):
    xw0 = _dot(x, wa)
    xw1 = _dot(x, wb)
    qv0 = _dot(xw0, q)
    qv1 = _dot(xw1, q)
    kv0 = _dot(xw0, k)
    kv1 = _dot(xw1, k)

    aggr_parts = []
    for b in range(B):
        lo = b * N
        q0b = qv0[lo:lo + N]
        q1b = qv1[lo:lo + N]
        k0r = jnp.transpose(kv0[lo:lo + N])
        k1r = jnp.transpose(kv1[lo:lo + N])
        pre = (q0b + rel1f * (q1b - q0b)) + (k0r + rel1f * (k1r - k0r))
        alpha = jnp.where(pre >= 0, pre, NEG_SLOPE * pre) + inval
        amax = jnp.max(alpha, axis=1, keepdims=True)
        ex = jnp.exp(alpha - amax)
        den = jnp.sum(ex, axis=1, keepdims=True)
        attn = ex / (den + 1e-16)
        aggr_parts.append(_dot(attn - attn * rel1f, xw0[lo:lo + N])
                          + _dot(attn * rel1f, xw1[lo:lo + N]))
    aggr = jnp.concatenate(aggr_parts, axis=0)

    msg1 = jnp.maximum(aggr + cb, 0.0)
    msg2 = _dot(x, lwa) + _dot(msg1, lwb) + lb
    mu = jnp.mean(msg2, axis=0, keepdims=True)
    var = jnp.mean((msg2 - mu) * (msg2 - mu), axis=0, keepdims=True)
    msg3 = (msg2 - mu) / jnp.sqrt(var + 1e-5) * gam + bet
    return x + msg3


def _rgat_body(x_ref,
               wa0, wb0, q0, k0, cb0, lwa0, lwb0, lb0, gam0, bet0,
               wa1, wb1, q1, k1, cb1, lwa1, lwb1, lb1, gam1, bet1,
               out_ref):
    d = jax.lax.broadcasted_iota(jnp.int32, (N, N), 0)
    s = jax.lax.broadcasted_iota(jnp.int32, (N, N), 1)
    rel1f = ((d // S0) != (s // S0)).astype(jnp.float32)
    inval = jnp.where(d == s, -1e30, 0.0)

    x = x_ref[...]
    x = _dense_layer(x, wa0[...], wb0[...], q0[...], k0[...], cb0[...],
                     lwa0[...], lwb0[...], lb0[...], gam0[...], bet0[...],
                     rel1f, inval)
    x = _dense_layer(x, wa1[...], wb1[...], q1[...], k1[...], cb1[...],
                     lwa1[...], lwb1[...], lb1[...], gam1[...], bet1[...],
                     rel1f, inval)
    out_ref[...] = x


def _run(x, args):
    return pl.pallas_call(
        _rgat_body,
        out_shape=jax.ShapeDtypeStruct((BN, F), jnp.float32),
    )(x, *args)


def _prep_layer(W, q, k, cb, lw, lb, gam, bet):
    return (W[0], W[1], q, k, cb.reshape(1, F),
            lw[:, :F].T, lw[:, F:].T, lb.reshape(1, F),
            gam.reshape(1, F), bet.reshape(1, F))


def kernel(desc0, desc1, W0, q0, k0, cb0, lw0, lb0, gam0, bet0,
           W1, q1, k1, cb1, lw1, lb1, gam1, bet1):
    x = jnp.concatenate([desc0, desc1], axis=2)
    x = jnp.transpose(x, (0, 2, 1)).reshape(BN, F)
    args = (_prep_layer(W0, q0, k0, cb0, lw0, lb0, gam0, bet0)
            + _prep_layer(W1, q1, k1, cb1, lw1, lb1, gam1, bet1))
    out = _run(x, args)
    out = jnp.transpose(out.reshape(B, N, F), (0, 2, 1))
    return out[:, :, :S0], out[:, :, S0:]


# speedup vs baseline: 2699.9414x; 1.7371x over previous
import jax
import jax.numpy as jnp
from jax.experimental import pallas as pl

B = 2
S0 = 256
S1 = 256
N = S0 + S1
F = 128
BN = B * N
NEG_SLOPE = 0.2


def _dot(a, b):
    return jax.lax.dot_general(
        a, b, (((1,), (0,)), ((), ())),
        precision=jax.lax.Precision.DEFAULT,
        preferred_element_type=jnp.float32)


def _dense_layer(x, wa, wb, q, k, cb, lwa, lwb, lb, gam, bet, diag_neg):
    xw = (_dot(x, wa), _dot(x, wb))
    qv = (_dot(xw[0], q), _dot(xw[1], q))
    kv = (_dot(xw[0], k), _dot(xw[1], k))

    aggr_parts = []
    for b in range(B):
        for g in range(2):
            r0 = b * N + g * S0
            pre = []
            for h in range(2):
                t = 0 if g == h else 1
                c0 = b * N + h * S0
                p = qv[t][r0:r0 + S0] + jnp.transpose(kv[t][c0:c0 + S0])
                p = jnp.maximum(p, NEG_SLOPE * p)
                if h == g:
                    p = p + diag_neg
                pre.append(p)
            amax = jnp.maximum(
                jnp.max(pre[0], axis=1, keepdims=True),
                jnp.max(pre[1], axis=1, keepdims=True))
            acc = None
            den = None
            for h in range(2):
                t = 0 if g == h else 1
                c0 = b * N + h * S0
                e = jnp.exp(pre[h] - amax)
                d = jnp.sum(e, axis=1, keepdims=True)
                a = _dot(e, xw[t][c0:c0 + S0])
                den = d if den is None else den + d
                acc = a if acc is None else acc + a
            aggr_parts.append(acc / (den + 1e-16))
    aggr = jnp.concatenate(aggr_parts, axis=0)

    msg1 = jnp.maximum(aggr + cb, 0.0)
    msg2 = _dot(x, lwa) + _dot(msg1, lwb) + lb
    mu = jnp.mean(msg2, axis=0, keepdims=True)
    var = jnp.mean((msg2 - mu) * (msg2 - mu), axis=0, keepdims=True)
    msg3 = (msg2 - mu) / jnp.sqrt(var + 1e-5) * gam + bet
    return x + msg3


def _rgat_body(x_ref,
               wa0, wb0, q0, k0, cb0, lwa0, lwb0, lb0, gam0, bet0,
               wa1, wb1, q1, k1, cb1, lwa1, lwb1, lb1, gam1, bet1,
               out_ref):
    d = jax.lax.broadcasted_iota(jnp.int32, (S0, S0), 0)
    s = jax.lax.broadcasted_iota(jnp.int32, (S0, S0), 1)
    diag_neg = jnp.where(d == s, -1e30, 0.0)

    x = x_ref[...]
    x = _dense_layer(x, wa0[...], wb0[...], q0[...], k0[...], cb0[...],
                     lwa0[...], lwb0[...], lb0[...], gam0[...], bet0[...],
                     diag_neg)
    x = _dense_layer(x, wa1[...], wb1[...], q1[...], k1[...], cb1[...],
                     lwa1[...], lwb1[...], lb1[...], gam1[...], bet1[...],
                     diag_neg)
    out_ref[...] = x


def _run(x, args):
    return pl.pallas_call(
        _rgat_body,
        out_shape=jax.ShapeDtypeStruct((BN, F), jnp.float32),
    )(x, *args)


def _prep_layer(W, q, k, cb, lw, lb, gam, bet):
    return (W[0], W[1], q, k, cb.reshape(1, F),
            lw[:, :F].T, lw[:, F:].T, lb.reshape(1, F),
            gam.reshape(1, F), bet.reshape(1, F))


def kernel(desc0, desc1, W0, q0, k0, cb0, lw0, lb0, gam0, bet0,
           W1, q1, k1, cb1, lw1, lb1, gam1, bet1):
    x = jnp.concatenate([desc0, desc1], axis=2)
    x = jnp.transpose(x, (0, 2, 1)).reshape(BN, F)
    args = (_prep_layer(W0, q0, k0, cb0, lw0, lb0, gam0, bet0)
            + _prep_layer(W1, q1, k1, cb1, lw1, lb1, gam1, bet1))
    out = _run(x, args)
    out = jnp.transpose(out.reshape(B, N, F), (0, 2, 1))
    return out[:, :, :S0], out[:, :, S0:]


# speedup vs baseline: 3103.0162x; 1.1493x over previous
import jax
import jax.numpy as jnp
from jax.experimental import pallas as pl

B = 2
S0 = 256
S1 = 256
N = S0 + S1
F = 128
BN = B * N
NEG_SLOPE = 0.2


def _mm(a, b):
    return jax.lax.dot_general(
        a, b, (((1,), (0,)), ((), ())),
        preferred_element_type=jnp.float32)


def _mm_tt(a, b):
    return jax.lax.dot_general(
        a, b, (((0,), (0,)), ((), ())),
        preferred_element_type=jnp.float32)


def _layer_t(xt, W, q, k, cb, lw, lb, gam, bet, diag_neg):
    wa, wb = W[0], W[1]
    xw = (_mm_tt(wa, xt), _mm_tt(wb, xt))
    qv = (_mm_tt(q, xw[0]), _mm_tt(q, xw[1]))
    kv = (_mm_tt(xw[0], k), _mm_tt(xw[1], k))

    aggr_parts = []
    for b in range(B):
        for g in range(2):
            d0 = b * N + g * S0
            pre = []
            for h in range(2):
                t = 0 if g == h else 1
                s0 = b * N + h * S0
                p = qv[t][:, d0:d0 + S0] + kv[t][s0:s0 + S0]
                p = jnp.maximum(p, NEG_SLOPE * p)
                if h == g:
                    p = p + diag_neg
                pre.append(p)
            amax = jnp.maximum(
                jnp.max(pre[0], axis=0, keepdims=True),
                jnp.max(pre[1], axis=0, keepdims=True))
            acc = None
            den = None
            for h in range(2):
                t = 0 if g == h else 1
                s0 = b * N + h * S0
                e = jnp.exp(pre[h] - amax)
                dsum = jnp.sum(e, axis=0, keepdims=True)
                a = _mm(xw[t][:, s0:s0 + S0], e)
                den = dsum if den is None else den + dsum
                acc = a if acc is None else acc + a
            aggr_parts.append(acc / (den + 1e-16))
    aggr = jnp.concatenate(aggr_parts, axis=1)

    msg1 = jnp.maximum(aggr + cb, 0.0)
    msg2 = _mm(lw[:, :F], xt) + _mm(lw[:, F:], msg1) + lb
    mu = jnp.mean(msg2, axis=1, keepdims=True)
    var = jnp.mean((msg2 - mu) * (msg2 - mu), axis=1, keepdims=True)
    msg3 = (msg2 - mu) / jnp.sqrt(var + 1e-5) * gam + bet
    return xt + msg3


def _rgat_body(d0_ref, d1_ref,
               W0, q0, k0, cb0, lw0, lb0, gam0, bet0,
               W1, q1, k1, cb1, lw1, lb1, gam1, bet1,
               o0_ref, o1_ref):
    di = jax.lax.broadcasted_iota(jnp.int32, (S0, S0), 0)
    si = jax.lax.broadcasted_iota(jnp.int32, (S0, S0), 1)
    diag_neg = jnp.where(di == si, -1e30, 0.0)

    xt = jnp.concatenate([d0_ref[0], d1_ref[0], d0_ref[1], d1_ref[1]], axis=1)
    xt = _layer_t(xt, W0[...], q0[...], k0[...], cb0[...], lw0[...],
                  lb0[...], gam0[...], bet0[...], diag_neg)
    xt = _layer_t(xt, W1[...], q1[...], k1[...], cb1[...], lw1[...],
                  lb1[...], gam1[...], bet1[...], diag_neg)
    for b in range(B):
        o0_ref[b] = xt[:, b * N:b * N + S0]
        o1_ref[b] = xt[:, b * N + S0:(b + 1) * N]


def kernel(desc0, desc1, W0, q0, k0, cb0, lw0, lb0, gam0, bet0,
           W1, q1, k1, cb1, lw1, lb1, gam1, bet1):
    return pl.pallas_call(
        _rgat_body,
        out_shape=(jax.ShapeDtypeStruct((B, F, S0), jnp.float32),
                   jax.ShapeDtypeStruct((B, F, S1), jnp.float32)),
    )(desc0, desc1,
      W0, q0, k0, cb0.reshape(F, 1), lw0, lb0.reshape(F, 1),
      gam0.reshape(F, 1), bet0.reshape(F, 1),
      W1, q1, k1, cb1.reshape(F, 1), lw1, lb1.reshape(F, 1),
      gam1.reshape(F, 1), bet1.reshape(F, 1))


# speedup vs baseline: 4898.0389x; 1.5785x over previous
import jax
import jax.numpy as jnp
from jax.experimental import pallas as pl

B = 2
S0 = 256
S1 = 256
N = S0 + S1
F = 128
BN = B * N
NEG_SLOPE = 0.2


def _mm(a, b):
    return jax.lax.dot_general(
        a, b, (((1,), (0,)), ((), ())),
        preferred_element_type=jnp.float32)


def _mm_tt(a, b):
    return jax.lax.dot_general(
        a, b, (((0,), (0,)), ((), ())),
        preferred_element_type=jnp.float32)


def _col(v):
    return jnp.reshape(v, (F, 1))


def _layer_t(xt, W, q, k, cb, lw, lb, gam, bet, diag_neg):
    wa, wb = W[0], W[1]
    cb, lb, gam, bet = _col(cb), _col(lb), _col(gam), _col(bet)
    xw = (_mm_tt(wa, xt), _mm_tt(wb, xt))
    qv = (_mm_tt(q, xw[0]), _mm_tt(q, xw[1]))
    kv = (_mm_tt(xw[0], k), _mm_tt(xw[1], k))

    aggr_parts = []
    for b in range(B):
        for g in range(2):
            d0 = b * N + g * S0
            pre = []
            for h in range(2):
                t = 0 if g == h else 1
                s0 = b * N + h * S0
                p = qv[t][:, d0:d0 + S0] + kv[t][s0:s0 + S0]
                p = jnp.maximum(p, NEG_SLOPE * p)
                if h == g:
                    p = p + diag_neg
                pre.append(p)
            amax = jnp.maximum(
                jnp.max(pre[0], axis=0, keepdims=True),
                jnp.max(pre[1], axis=0, keepdims=True))
            acc = None
            den = None
            for h in range(2):
                t = 0 if g == h else 1
                s0 = b * N + h * S0
                e = jnp.exp(pre[h] - amax)
                dsum = jnp.sum(e, axis=0, keepdims=True)
                a = _mm(xw[t][:, s0:s0 + S0], e)
                den = dsum if den is None else den + dsum
                acc = a if acc is None else acc + a
            aggr_parts.append(acc / (den + 1e-16))
    aggr = jnp.concatenate(aggr_parts, axis=1)

    msg1 = jnp.maximum(aggr + cb, 0.0)
    msg2 = _mm(lw[:, :F], xt) + _mm(lw[:, F:], msg1) + lb
    mu = jnp.mean(msg2, axis=1, keepdims=True)
    var = jnp.mean((msg2 - mu) * (msg2 - mu), axis=1, keepdims=True)
    msg3 = (msg2 - mu) / jnp.sqrt(var + 1e-5) * gam + bet
    return xt + msg3


def _rgat_body(d0_ref, d1_ref,
               W0, q0, k0, cb0, lw0, lb0, gam0, bet0,
               W1, q1, k1, cb1, lw1, lb1, gam1, bet1,
               o0_ref, o1_ref):
    di = jax.lax.broadcasted_iota(jnp.int32, (S0, S0), 0)
    si = jax.lax.broadcasted_iota(jnp.int32, (S0, S0), 1)
    diag_neg = jnp.where(di == si, -1e30, 0.0)

    xt = jnp.concatenate([d0_ref[0], d1_ref[0], d0_ref[1], d1_ref[1]], axis=1)
    xt = _layer_t(xt, W0[...], q0[...], k0[...], cb0[...], lw0[...],
                  lb0[...], gam0[...], bet0[...], diag_neg)
    xt = _layer_t(xt, W1[...], q1[...], k1[...], cb1[...], lw1[...],
                  lb1[...], gam1[...], bet1[...], diag_neg)
    for b in range(B):
        o0_ref[b] = xt[:, b * N:b * N + S0]
        o1_ref[b] = xt[:, b * N + S0:(b + 1) * N]


def kernel(desc0, desc1, W0, q0, k0, cb0, lw0, lb0, gam0, bet0,
           W1, q1, k1, cb1, lw1, lb1, gam1, bet1):
    return pl.pallas_call(
        _rgat_body,
        out_shape=(jax.ShapeDtypeStruct((B, F, S0), jnp.float32),
                   jax.ShapeDtypeStruct((B, F, S1), jnp.float32)),
    )(desc0, desc1,
      W0, q0, k0, cb0, lw0, lb0, gam0, bet0,
      W1, q1, k1, cb1, lw1, lb1, gam1, bet1)
